# Initial kernel scaffold; baseline (speedup 1.0000x reference)
import jax
import jax.numpy as jnp
from jax.experimental import pallas as pl


def kernel(cost_volume, peak_coords):
    raise NotImplementedError("write your pallas kernel here")


# speedup vs baseline: 3.9369x; 3.9369x over previous
import functools

import jax
import jax.numpy as jnp
from jax.experimental import pallas as pl

_B, _C, _H, _W = 16, 1024, 64, 64
_CB = 128
_NCB = _C // _CB
_R = 2


def _psr_kernel(cv_ref, pc_ref, out_ref):
    cb = pl.program_id(1)

    v = cv_ref[0]
    pcs = pc_ref[0, pl.ds(cb * _CB, _CB), :]
    py = pcs[:, 0:1]
    px = pcs[:, 1:2]

    ih = jax.lax.broadcasted_iota(jnp.int32, (_CB, _H), 1)
    iw = jax.lax.broadcasted_iota(jnp.int32, (_CB, _W), 1)
    hmask = (jnp.abs(ih - py) <= _R).astype(jnp.float32)
    wmask = (jnp.abs(iw - px) <= _R).astype(jnp.float32)
    wside = 1.0 - hmask[:, :, None] * wmask[:, None, :]

    n = _H * _W - jnp.sum(hmask, axis=1, keepdims=True) * jnp.sum(
        wmask, axis=1, keepdims=True
    )
    sv = jnp.sum(v * wside, axis=(1, 2), keepdims=True)[:, :, 0]
    mean = sv / n
    dv = (v - mean[:, :, None]) * wside
    var = jnp.sum(dv * dv, axis=(1, 2), keepdims=True)[:, :, 0] / (n - 1.0)
    peak = jnp.max(v, axis=(1, 2), keepdims=True)[:, :, 0]
    psr = (peak - mean) / var

    out_ref[0, pl.ds(cb * _CB, _CB), :] = psr

    @pl.when(cb == _NCB - 1)
    def _():
        allp = out_ref[0, :, :]
        m = jnp.sum(allp) / _C
        out_ref[0, :, :] = allp / (m + 1e-08)


def kernel(cost_volume, peak_coords):
    pc = peak_coords.astype(jnp.int32)
    out = pl.pallas_call(
        _psr_kernel,
        grid=(_B, _NCB),
        in_specs=[
            pl.BlockSpec((1, _CB, _H, _W), lambda b, cb: (b, cb, 0, 0)),
            pl.BlockSpec((1, _C, 2), lambda b, cb: (b, 0, 0)),
        ],
        out_specs=pl.BlockSpec((1, _C, 1), lambda b, cb: (b, 0, 0)),
        out_shape=jax.ShapeDtypeStruct((_B, _C, 1), jnp.float32),
    )(cost_volume, pc)
    return out.reshape(_B, _C)
